# baseline (speedup 1.0000x reference)
import math

import jax
import jax.numpy as jnp
from jax.experimental import pallas as pl
from jax.experimental.pallas import tpu as pltpu


def kernel(hidden_states, Wq, bq, Wk, bk, Wv, bv, Wo, bo,
           memory_keys, memory_values, memory_age):
    batch, seq, hidden = hidden_states.shape
    heads, msize, _ = memory_keys.shape
    f32 = jnp.float32
    scale = 1.0 / math.sqrt(hidden)

    hs = hidden_states.reshape(batch, hidden)
    K2 = memory_keys.reshape(msize, hidden)
    V2 = memory_values.reshape(msize, hidden)
    age2 = memory_age.reshape(1, msize)

    BM = 2048
    NB = msize // BM
    dn_nt = (((1,), (1,)), ((), ()))
    dn_nn = (((1,), (0,)), ((), ()))

    def _proj(hs_ref, wq_ref, bq_ref, wk_ref, bk_ref, wv_ref, bv_ref,
              q_ref, k_ref, v_ref):
        x = hs_ref[...]
        q_ref[...] = jax.lax.dot_general(x, wq_ref[...], dn_nt,
                                         preferred_element_type=f32) + bq_ref[...]
        k_ref[...] = jax.lax.dot_general(x, wk_ref[...], dn_nt,
                                         preferred_element_type=f32) + bk_ref[...]
        v_ref[...] = jax.lax.dot_general(x, wv_ref[...], dn_nt,
                                         preferred_element_type=f32) + bv_ref[...]

    q, kproj, vproj = pl.pallas_call(
        _proj,
        out_shape=[jax.ShapeDtypeStruct((batch, hidden), f32)] * 3,
    )(hs, Wq, bq.reshape(1, hidden), Wk, bk.reshape(1, hidden),
      Wv, bv.reshape(1, hidden))

    def _scores(q_ref, k_ref, s_ref, nk_ref):
        kblk = k_ref[...]
        s = jax.lax.dot_general(q_ref[...], kblk, dn_nt,
                                preferred_element_type=f32)
        s_ref[...] = s * scale
        nk_ref[...] = kblk

    scores, new_keys2 = pl.pallas_call(
        _scores,
        grid=(NB,),
        in_specs=[pl.BlockSpec((batch, hidden), lambda i: (0, 0)),
                  pl.BlockSpec((BM, hidden), lambda i: (i, 0))],
        out_specs=[pl.BlockSpec((batch, BM), lambda i: (0, i)),
                   pl.BlockSpec((BM, hidden), lambda i: (i, 0))],
        out_shape=[jax.ShapeDtypeStruct((batch, msize), f32),
                   jax.ShapeDtypeStruct((msize, hidden), f32)],
    )(q, K2)

    def _stats(s_ref, m_ref, l_ref, ms_ref):
        s = s_ref[...]
        m = jnp.max(s, axis=1, keepdims=True)
        l = jnp.sum(jnp.exp(s - m), axis=1, keepdims=True)
        m_ref[...] = jnp.broadcast_to(m, m_ref.shape)
        l_ref[...] = jnp.broadcast_to(l, l_ref.shape)
        ms_ref[...] = jnp.full(ms_ref.shape, jnp.mean(m), f32)

    mrow, lrow, msarr = pl.pallas_call(
        _stats,
        out_shape=[jax.ShapeDtypeStruct((batch, 128), f32)] * 3,
    )(scores)

    def _values(s_ref, m_ref, l_ref, v_ref, age_ref,
                nv_ref, o_ref, ac_ref, na_ref, idx_ref, usage_ref,
                best_ref, bint_ref):
        i = pl.program_id(0)

        @pl.when(i == 0)
        def _():
            o_ref[...] = jnp.zeros_like(o_ref)
            best_ref[0] = -jnp.inf
            best_ref[1] = 0.0
            bint_ref[0] = 0
            bint_ref[1] = 0

        m = m_ref[:, 0:1]
        l = l_ref[:, 0:1]
        p = jnp.exp(s_ref[...] - m) / l
        v = v_ref[...]
        nv_ref[...] = v
        o_ref[...] += jax.lax.dot_general(p, v, dn_nn,
                                          preferred_element_type=f32)

        na = age_ref[...] + 1.0
        na_ref[...] = na
        imp = jnp.sum(p, axis=0, keepdims=True)
        ac_ref[...] = jnp.sum((p > 0.01).astype(jnp.int32), axis=0,
                              keepdims=True)

        t = na + (1.0 - imp)
        col = jax.lax.broadcasted_iota(jnp.int32, t.shape, 1)
        tmax = jnp.max(t)
        amax = jnp.min(jnp.where(t == tmax, col, t.shape[1]))
        na_at = jnp.sum(jnp.where(col == amax, na, 0.0))
        npos = jnp.sum((na > 0.0).astype(jnp.int32))

        prev = best_ref[0]
        better = tmax > prev
        best_ref[0] = jnp.where(better, tmax, prev)
        best_ref[1] = jnp.where(better, na_at, best_ref[1])
        bint_ref[0] = jnp.where(better, i * BM + amax, bint_ref[0])
        bint_ref[1] = bint_ref[1] + npos

        @pl.when(i == NB - 1)
        def _():
            idx_ref[...] = jnp.full(idx_ref.shape, bint_ref[0], jnp.int32)
            zeroed = (best_ref[1] > 0.0).astype(f32)
            usage_ref[...] = jnp.full(
                usage_ref.shape,
                (bint_ref[1].astype(f32) - zeroed) / msize, f32)

    new_values2, o_acc, ac_row, na_row, idx_out, usage_out = pl.pallas_call(
        _values,
        grid=(NB,),
        in_specs=[pl.BlockSpec((batch, BM), lambda i: (0, i)),
                  pl.BlockSpec((batch, 128), lambda i: (0, 0)),
                  pl.BlockSpec((batch, 128), lambda i: (0, 0)),
                  pl.BlockSpec((BM, hidden), lambda i: (i, 0)),
                  pl.BlockSpec((1, BM), lambda i: (0, i))],
        out_specs=[pl.BlockSpec((BM, hidden), lambda i: (i, 0)),
                   pl.BlockSpec((batch, hidden), lambda i: (0, 0)),
                   pl.BlockSpec((1, BM), lambda i: (0, i)),
                   pl.BlockSpec((1, BM), lambda i: (0, i)),
                   pl.BlockSpec((batch, 128), lambda i: (0, 0)),
                   pl.BlockSpec((batch, 128), lambda i: (0, 0))],
        out_shape=[jax.ShapeDtypeStruct((msize, hidden), f32),
                   jax.ShapeDtypeStruct((batch, hidden), f32),
                   jax.ShapeDtypeStruct((1, msize), jnp.int32),
                   jax.ShapeDtypeStruct((1, msize), f32),
                   jax.ShapeDtypeStruct((batch, 128), jnp.int32),
                   jax.ShapeDtypeStruct((batch, 128), f32)],
        scratch_shapes=[pltpu.SMEM((2,), f32), pltpu.SMEM((2,), jnp.int32)],
    )(scores, mrow, lrow, V2, age2)

    def _outproj(o_ref, wo_ref, bo_ref, y_ref):
        y_ref[...] = jax.lax.dot_general(o_ref[...], wo_ref[...], dn_nt,
                                         preferred_element_type=f32) + bo_ref[...]

    out = pl.pallas_call(
        _outproj,
        out_shape=jax.ShapeDtypeStruct((batch, hidden), f32),
    )(o_acc, Wo, bo.reshape(1, hidden))

    idx1 = idx_out[0, 0:1]
    updk = kproj[0:1].reshape(1, 1, hidden)
    updv = vproj[0:1].reshape(1, 1, hidden)
    nk3 = new_keys2.reshape(msize, 1, hidden)
    nv3 = new_values2.reshape(msize, 1, hidden)
    na3 = na_row.reshape(msize // 128, 1, 128)

    def _scatter(idx_ref, updk_ref, updv_ref, kin_ref, vin_ref, ain_ref,
                 kout_ref, vout_ref, aout_ref):
        del kin_ref, vin_ref
        kout_ref[...] = updk_ref[...]
        vout_ref[...] = updv_ref[...]
        lane = idx_ref[0] % 128
        colv = jax.lax.broadcasted_iota(jnp.int32, aout_ref.shape, 2)
        aout_ref[...] = jnp.where(colv == lane, 0.0, ain_ref[...])

    grid_spec = pltpu.PrefetchScalarGridSpec(
        num_scalar_prefetch=1,
        grid=(1,),
        in_specs=[
            pl.BlockSpec((1, 1, hidden), lambda i, idx: (0, 0, 0)),
            pl.BlockSpec((1, 1, hidden), lambda i, idx: (0, 0, 0)),
            pl.BlockSpec((1, 1, hidden), lambda i, idx: (idx[0], 0, 0)),
            pl.BlockSpec((1, 1, hidden), lambda i, idx: (idx[0], 0, 0)),
            pl.BlockSpec((1, 1, 128), lambda i, idx: (idx[0] // 128, 0, 0)),
        ],
        out_specs=[
            pl.BlockSpec((1, 1, hidden), lambda i, idx: (idx[0], 0, 0)),
            pl.BlockSpec((1, 1, hidden), lambda i, idx: (idx[0], 0, 0)),
            pl.BlockSpec((1, 1, 128), lambda i, idx: (idx[0] // 128, 0, 0)),
        ],
    )
    nk_f, nv_f, na_f = pl.pallas_call(
        _scatter,
        grid_spec=grid_spec,
        out_shape=[jax.ShapeDtypeStruct((msize, 1, hidden), f32),
                   jax.ShapeDtypeStruct((msize, 1, hidden), f32),
                   jax.ShapeDtypeStruct((msize // 128, 1, 128), f32)],
        input_output_aliases={3: 0, 4: 1, 5: 2},
    )(idx1, updk, updv, nk3, nv3, na3)

    output = out.reshape(batch, seq, hidden)
    access_counts = ac_row.reshape(heads, msize)
    max_scores = msarr[0, 0]
    memory_usage = usage_out[0, 0]
    new_keys = nk_f.reshape(heads, msize, hidden)
    new_values = nv_f.reshape(heads, msize, hidden)
    new_age = na_f.reshape(heads, msize)
    return (output, access_counts, max_scores, memory_usage,
            new_keys, new_values, new_age)


# speedup vs baseline: 3.2222x; 3.2222x over previous
import math

import jax
import jax.numpy as jnp
from jax.experimental import pallas as pl
from jax.experimental.pallas import tpu as pltpu


def kernel(hidden_states, Wq, bq, Wk, bk, Wv, bv, Wo, bo,
           memory_keys, memory_values, memory_age):
    batch, seq, hidden = hidden_states.shape
    heads, msize, _ = memory_keys.shape
    f32 = jnp.float32
    scale = 1.0 / math.sqrt(hidden)

    hs = hidden_states.reshape(batch, hidden)
    K2 = memory_keys.reshape(msize, hidden)
    V2 = memory_values.reshape(msize, hidden)
    age2 = memory_age.reshape(1, msize)

    BM = 4096
    NB = msize // BM
    dn_nt = (((1,), (1,)), ((), ()))
    dn_nn = (((1,), (0,)), ((), ()))

    def _kstream(hs_ref, wq_ref, bq_ref, wk_ref, bk_ref, wv_ref, bv_ref,
                 k_ref,
                 kp_ref, vp_ref, s_ref, nk_ref, m_ref, l_ref, ms_ref,
                 q_scr, m_scr, l_scr):
        i = pl.program_id(0)

        @pl.when(i == 0)
        def _():
            x = hs_ref[...]
            q_scr[...] = jax.lax.dot_general(
                x, wq_ref[...], dn_nt, preferred_element_type=f32) + bq_ref[...]
            kp_ref[...] = jax.lax.dot_general(
                x, wk_ref[...], dn_nt, preferred_element_type=f32) + bk_ref[...]
            vp_ref[...] = jax.lax.dot_general(
                x, wv_ref[...], dn_nt, preferred_element_type=f32) + bv_ref[...]
            m_scr[...] = jnp.full(m_scr.shape, -jnp.inf, f32)
            l_scr[...] = jnp.zeros(l_scr.shape, f32)

        kblk = k_ref[...]
        s = jax.lax.dot_general(q_scr[...], kblk, dn_nt,
                                preferred_element_type=f32) * scale
        s_ref[...] = s
        nk_ref[...] = kblk

        bmax = jnp.max(s, axis=1, keepdims=True)
        m_old = m_scr[...]
        m_new = jnp.maximum(m_old, bmax)
        l_new = (l_scr[...] * jnp.exp(m_old - m_new)
                 + jnp.sum(jnp.exp(s - bmax), axis=1, keepdims=True)
                 * jnp.exp(bmax - m_new))
        m_scr[...] = m_new
        l_scr[...] = l_new

        @pl.when(i == NB - 1)
        def _():
            m_ref[...] = m_new
            l_ref[...] = l_new
            ms_ref[...] = jnp.full(ms_ref.shape, jnp.mean(m_new[:, 0:1]), f32)

    kproj, vproj, scores, new_keys2, mrow, lrow, msarr = pl.pallas_call(
        _kstream,
        grid=(NB,),
        in_specs=[pl.BlockSpec((batch, hidden), lambda i: (0, 0)),
                  pl.BlockSpec((hidden, hidden), lambda i: (0, 0)),
                  pl.BlockSpec((1, hidden), lambda i: (0, 0)),
                  pl.BlockSpec((hidden, hidden), lambda i: (0, 0)),
                  pl.BlockSpec((1, hidden), lambda i: (0, 0)),
                  pl.BlockSpec((hidden, hidden), lambda i: (0, 0)),
                  pl.BlockSpec((1, hidden), lambda i: (0, 0)),
                  pl.BlockSpec((BM, hidden), lambda i: (i, 0))],
        out_specs=[pl.BlockSpec((batch, hidden), lambda i: (0, 0)),
                   pl.BlockSpec((batch, hidden), lambda i: (0, 0)),
                   pl.BlockSpec((batch, BM), lambda i: (0, i)),
                   pl.BlockSpec((BM, hidden), lambda i: (i, 0)),
                   pl.BlockSpec((batch, 128), lambda i: (0, 0)),
                   pl.BlockSpec((batch, 128), lambda i: (0, 0)),
                   pl.BlockSpec((batch, 128), lambda i: (0, 0))],
        out_shape=[jax.ShapeDtypeStruct((batch, hidden), f32),
                   jax.ShapeDtypeStruct((batch, hidden), f32),
                   jax.ShapeDtypeStruct((batch, msize), f32),
                   jax.ShapeDtypeStruct((msize, hidden), f32),
                   jax.ShapeDtypeStruct((batch, 128), f32),
                   jax.ShapeDtypeStruct((batch, 128), f32),
                   jax.ShapeDtypeStruct((batch, 128), f32)],
        scratch_shapes=[pltpu.VMEM((batch, hidden), f32),
                        pltpu.VMEM((batch, 128), f32),
                        pltpu.VMEM((batch, 128), f32)],
    )(hs, Wq, bq.reshape(1, hidden), Wk, bk.reshape(1, hidden),
      Wv, bv.reshape(1, hidden), K2)

    def _vstream(s_ref, m_ref, l_ref, v_ref, age_ref, wo_ref, bo_ref,
                 nv_ref, y_ref, ac_ref, na_ref, idx_ref, usage_ref,
                 o_scr, best_ref, bint_ref):
        i = pl.program_id(0)

        @pl.when(i == 0)
        def _():
            o_scr[...] = jnp.zeros(o_scr.shape, f32)
            best_ref[0] = -jnp.inf
            best_ref[1] = 0.0
            bint_ref[0] = 0
            bint_ref[1] = 0

        m = m_ref[:, 0:1]
        l = l_ref[:, 0:1]
        p = jnp.exp(s_ref[...] - m) / l
        v = v_ref[...]
        nv_ref[...] = v
        o_scr[...] += jax.lax.dot_general(p, v, dn_nn,
                                          preferred_element_type=f32)

        na = age_ref[...] + 1.0
        na_ref[...] = na
        imp = jnp.sum(p, axis=0, keepdims=True)
        ac_ref[...] = jnp.sum((p > 0.01).astype(jnp.int32), axis=0,
                              keepdims=True)

        t = na + (1.0 - imp)
        col = jax.lax.broadcasted_iota(jnp.int32, t.shape, 1)
        tmax = jnp.max(t)
        amax = jnp.min(jnp.where(t == tmax, col, t.shape[1]))
        na_at = jnp.sum(jnp.where(col == amax, na, 0.0))
        npos = jnp.sum((na > 0.0).astype(jnp.int32))

        prev = best_ref[0]
        better = tmax > prev
        best_ref[0] = jnp.where(better, tmax, prev)
        best_ref[1] = jnp.where(better, na_at, best_ref[1])
        bint_ref[0] = jnp.where(better, i * BM + amax, bint_ref[0])
        bint_ref[1] = bint_ref[1] + npos

        @pl.when(i == NB - 1)
        def _():
            y_ref[...] = jax.lax.dot_general(
                o_scr[...], wo_ref[...], dn_nt,
                preferred_element_type=f32) + bo_ref[...]
            idx_ref[...] = jnp.full(idx_ref.shape, bint_ref[0], jnp.int32)
            zeroed = (best_ref[1] > 0.0).astype(f32)
            usage_ref[...] = jnp.full(
                usage_ref.shape,
                (bint_ref[1].astype(f32) - zeroed) / msize, f32)

    new_values2, out, ac_row, na_row, idx_out, usage_out = pl.pallas_call(
        _vstream,
        grid=(NB,),
        in_specs=[pl.BlockSpec((batch, BM), lambda i: (0, i)),
                  pl.BlockSpec((batch, 128), lambda i: (0, 0)),
                  pl.BlockSpec((batch, 128), lambda i: (0, 0)),
                  pl.BlockSpec((BM, hidden), lambda i: (i, 0)),
                  pl.BlockSpec((1, BM), lambda i: (0, i)),
                  pl.BlockSpec((hidden, hidden), lambda i: (0, 0)),
                  pl.BlockSpec((1, hidden), lambda i: (0, 0))],
        out_specs=[pl.BlockSpec((BM, hidden), lambda i: (i, 0)),
                   pl.BlockSpec((batch, hidden), lambda i: (0, 0)),
                   pl.BlockSpec((1, BM), lambda i: (0, i)),
                   pl.BlockSpec((1, BM), lambda i: (0, i)),
                   pl.BlockSpec((batch, 128), lambda i: (0, 0)),
                   pl.BlockSpec((batch, 128), lambda i: (0, 0))],
        out_shape=[jax.ShapeDtypeStruct((msize, hidden), f32),
                   jax.ShapeDtypeStruct((batch, hidden), f32),
                   jax.ShapeDtypeStruct((1, msize), jnp.int32),
                   jax.ShapeDtypeStruct((1, msize), f32),
                   jax.ShapeDtypeStruct((batch, 128), jnp.int32),
                   jax.ShapeDtypeStruct((batch, 128), f32)],
        scratch_shapes=[pltpu.VMEM((batch, hidden), f32),
                        pltpu.SMEM((2,), f32), pltpu.SMEM((2,), jnp.int32)],
    )(scores, mrow, lrow, V2, age2, Wo, bo.reshape(1, hidden))

    idx1 = idx_out[0, 0:1]
    updk = kproj[0:1]
    updv = vproj[0:1]

    def _scatter(idx_ref, updk_ref, updv_ref, kin_ref, vin_ref, ain_ref,
                 kout_ref, vout_ref, aout_ref):
        row = idx_ref[0] % 8
        lane = idx_ref[0] % 128
        rowv = jax.lax.broadcasted_iota(jnp.int32, kin_ref.shape, 0)
        kout_ref[...] = jnp.where(rowv == row, updk_ref[...], kin_ref[...])
        vout_ref[...] = jnp.where(rowv == row, updv_ref[...], vin_ref[...])
        colv = jax.lax.broadcasted_iota(jnp.int32, ain_ref.shape, 1)
        aout_ref[...] = jnp.where(colv == lane, 0.0, ain_ref[...])

    grid_spec = pltpu.PrefetchScalarGridSpec(
        num_scalar_prefetch=1,
        grid=(1,),
        in_specs=[
            pl.BlockSpec((1, hidden), lambda i, idx: (0, 0)),
            pl.BlockSpec((1, hidden), lambda i, idx: (0, 0)),
            pl.BlockSpec((8, hidden), lambda i, idx: (idx[0] // 8, 0)),
            pl.BlockSpec((8, hidden), lambda i, idx: (idx[0] // 8, 0)),
            pl.BlockSpec((1, 128), lambda i, idx: (0, idx[0] // 128)),
        ],
        out_specs=[
            pl.BlockSpec((8, hidden), lambda i, idx: (idx[0] // 8, 0)),
            pl.BlockSpec((8, hidden), lambda i, idx: (idx[0] // 8, 0)),
            pl.BlockSpec((1, 128), lambda i, idx: (0, idx[0] // 128)),
        ],
    )
    nk_f, nv_f, na_f = pl.pallas_call(
        _scatter,
        grid_spec=grid_spec,
        out_shape=[jax.ShapeDtypeStruct((msize, hidden), f32),
                   jax.ShapeDtypeStruct((msize, hidden), f32),
                   jax.ShapeDtypeStruct((1, msize), f32)],
        input_output_aliases={3: 0, 4: 1, 5: 2},
    )(idx1, updk, updv, new_keys2, new_values2, na_row)

    output = out.reshape(batch, seq, hidden)
    access_counts = ac_row.reshape(heads, msize)
    max_scores = msarr[0, 0]
    memory_usage = usage_out[0, 0]
    new_keys = nk_f.reshape(heads, msize, hidden)
    new_values = nv_f.reshape(heads, msize, hidden)
    new_age = na_f.reshape(heads, msize)
    return (output, access_counts, max_scores, memory_usage,
            new_keys, new_values, new_age)
